# Initial kernel scaffold; baseline (speedup 1.0000x reference)
import functools

import jax
import jax.numpy as jnp
from jax import lax
from jax.experimental import pallas as pl
from jax.experimental.pallas import tpu as pltpu
from jax.experimental.pallas import tpu_sc as plsc

N = 10000
D = 128
T = 64
G = 64

NPAD = 10016
C = 128
NCHUNK = 80
NW = 32
EP = NW * NCHUNK * C
ZROWS = NPAD // 16
OROWS = N // 16

_mesh = plsc.VectorSubcoreMesh(core_axis_name="c", subcore_axis_name="s")


def _make_agg(d, with_cnt):
    out_types = [jax.ShapeDtypeStruct((2 * N, d), jnp.float32)]
    scratch = [
        pltpu.VMEM((NCHUNK, C), jnp.int32),
        pltpu.VMEM((NCHUNK, C), jnp.int32),
        pltpu.VMEM((C, d), jnp.float32),
        pltpu.VMEM((C, d), jnp.float32),
        pltpu.VMEM_SHARED((NPAD, d), jnp.float32),
        pltpu.SemaphoreType.DMA,
        pltpu.SemaphoreType.DMA,
    ]
    if with_cnt:
        out_types.append(jax.ShapeDtypeStruct((2 * N, 16), jnp.float32))
        scratch += [
            pltpu.VMEM((C, 16), jnp.float32),
            pltpu.VMEM_SHARED((NPAD, 16), jnp.float32),
        ]

    @functools.partial(
        pl.kernel,
        out_type=out_types if with_cnt else out_types[0],
        mesh=_mesh,
        scratch_types=scratch,
    )
    def agg(table_hbm, src_hbm, dst_hbm, *rest):
        if with_cnt:
            (out_hbm, cnt_hbm, src_v, dst_v, bufa, bufb, acc,
             sga, sgb, ones_v, acc_cnt) = rest
        else:
            out_hbm, src_v, dst_v, bufa, bufb, acc, sga, sgb = rest
        cid = lax.axis_index("c")
        sid = lax.axis_index("s")
        wid = cid * 16 + sid

        pltpu.sync_copy(src_hbm.at[wid], src_v)
        pltpu.sync_copy(dst_hbm.at[wid], dst_v)

        @pl.loop(0, C)
        def _(r):
            @pl.loop(0, d, step=16)
            def _(cc):
                bufa[r, pl.ds(cc, 16)] = jnp.zeros((16,), jnp.float32)

        r0 = sid * ZROWS
        nfull = ZROWS // C
        for k in range(nfull):
            pltpu.sync_copy(bufa, acc.at[pl.ds(r0 + k * C, C)])
        rem = ZROWS - nfull * C
        if rem:
            pltpu.sync_copy(bufa.at[pl.ds(0, rem)],
                            acc.at[pl.ds(r0 + nfull * C, rem)])

        if with_cnt:
            @pl.loop(0, C)
            def _(r):
                ones_v[r, :] = jnp.zeros((16,), jnp.float32)

            for k in range(nfull):
                pltpu.sync_copy(ones_v, acc_cnt.at[pl.ds(r0 + k * C, C)])
            if rem:
                pltpu.sync_copy(ones_v.at[pl.ds(0, rem)],
                                acc_cnt.at[pl.ds(r0 + nfull * C, rem)])

            @pl.loop(0, C)
            def _(r):
                ones_v[r, :] = jnp.ones((16,), jnp.float32)

        plsc.subcore_barrier()

        def start_gather(j, buf, sem):
            pltpu.make_async_copy(table_hbm.at[src_v.at[j]], buf, sem).start()

        def wait_gather(j, buf, sem):
            pltpu.make_async_copy(table_hbm.at[src_v.at[j]], buf, sem).wait()

        def scatter_add(j, buf):
            pltpu.sync_copy(buf, acc.at[dst_v.at[j]], add=True)
            if with_cnt:
                pltpu.sync_copy(ones_v, acc_cnt.at[dst_v.at[j]], add=True)

        start_gather(0, bufa, sga)

        @pl.loop(0, NCHUNK, step=2)
        def _(j):
            wait_gather(j, bufa, sga)
            start_gather(j + 1, bufb, sgb)
            scatter_add(j, bufa)
            wait_gather(j + 1, bufb, sgb)

            @pl.when(j + 2 < NCHUNK)
            def _():
                start_gather(j + 2, bufa, sga)

            scatter_add(j + 1, bufb)

        plsc.subcore_barrier()

        ro = sid * OROWS
        pltpu.sync_copy(acc.at[pl.ds(ro, OROWS)],
                        out_hbm.at[pl.ds(cid * N + ro, OROWS)])
        if with_cnt:
            pltpu.sync_copy(acc_cnt.at[pl.ds(ro, OROWS)],
                            cnt_hbm.at[pl.ds(cid * N + ro, OROWS)])

    return agg


_agg_cnt = _make_agg(D, True)
_agg128 = _make_agg(D, False)
_agg64 = _make_agg(T, False)


def _mean_from_partials(p_ref, cnt_ref):
    psum = p_ref[0:N, :] + p_ref[N:2 * N, :]
    cnt = cnt_ref[0:N, 0:1] + cnt_ref[N:2 * N, 0:1]
    return psum / jnp.maximum(cnt, 1.0)


def _dense_body(p_ref, cnt_ref, h_ref, wl_ref, wr_ref, b_ref, g_ref, be_ref,
                o_ref):
    mean = _mean_from_partials(p_ref, cnt_ref)
    z = (jnp.dot(mean, wl_ref[...], preferred_element_type=jnp.float32)
         + jnp.dot(h_ref[...], wr_ref[...], preferred_element_type=jnp.float32)
         + b_ref[...])
    mu = jnp.mean(z, axis=0, keepdims=True)
    var = jnp.mean((z - mu) * (z - mu), axis=0, keepdims=True)
    zn = (z - mu) / jnp.sqrt(var + 1e-5) * g_ref[...] + be_ref[...]
    o_ref[...] = jnp.maximum(zn, 0.0)


def _dense_proj_body(p_ref, cnt_ref, h_ref, wl_ref, wr_ref, b_ref, g_ref,
                     be_ref, wlo_ref, o_ref, y_ref):
    _dense_body(p_ref, cnt_ref, h_ref, wl_ref, wr_ref, b_ref, g_ref, be_ref,
                o_ref)
    y_ref[...] = jnp.dot(o_ref[...], wlo_ref[...],
                         preferred_element_type=jnp.float32)


def _final_body(p_ref, cnt_ref, h_ref, batch_ref, wro_ref, bo_ref, o_ref):
    s = _mean_from_partials(p_ref, cnt_ref)
    gi = lax.broadcasted_iota(jnp.int32, (G, N), 0)
    bm = (batch_ref[0:1, :] == gi).astype(jnp.float32)
    cg = jnp.sum(bm, axis=1, keepdims=True)
    pw = bm * jnp.where(cg > 0, 1.0 / jnp.maximum(cg, 1.0), 0.0)
    pooled_s = jnp.dot(pw, s, preferred_element_type=jnp.float32)
    pooled_h = jnp.dot(pw, h_ref[...], preferred_element_type=jnp.float32)
    o = (pooled_s
         + jnp.dot(pooled_h, wro_ref[...], preferred_element_type=jnp.float32)
         + bo_ref[...])
    o_ref[...] = jnp.where(cg > 0, o, 0.0)


def _dense(p, cntp, h, wl, wr, b, g, be):
    return pl.pallas_call(
        _dense_body,
        out_shape=jax.ShapeDtypeStruct((N, D), jnp.float32),
    )(p, cntp, h, wl, wr, b.reshape(1, -1), g.reshape(1, -1),
      be.reshape(1, -1))


def _dense_proj(p, cntp, h, wl, wr, b, g, be, wlo):
    return pl.pallas_call(
        _dense_proj_body,
        out_shape=[jax.ShapeDtypeStruct((N, D), jnp.float32),
                   jax.ShapeDtypeStruct((N, T), jnp.float32)],
    )(p, cntp, h, wl, wr, b.reshape(1, -1), g.reshape(1, -1),
      be.reshape(1, -1), wlo)


def _final(p, cntp, h, batch2d, wro, bo):
    return pl.pallas_call(
        _final_body,
        out_shape=jax.ShapeDtypeStruct((G, T), jnp.float32),
    )(p, cntp, h, batch2d, wro, bo.reshape(1, -1))


def kernel(x, edge_index, batch, Wl0, Wr0, b0, g1, be1, Wl1, Wr1, b1, g2, be2,
           Wl2, Wr2, b2, g3, be3, Wlo, Wro, bo):
    e = edge_index.shape[1]
    pad = EP - e
    src = jnp.concatenate(
        [edge_index[0], jnp.zeros((pad,), jnp.int32)]).reshape(NW, NCHUNK, C)
    dst = jnp.concatenate(
        [edge_index[1], jnp.full((pad,), N, jnp.int32)]).reshape(NW, NCHUNK, C)

    p0, cntp = _agg_cnt(x, src, dst)
    h1 = _dense(p0, cntp, x, Wl0, Wr0, b0, g1, be1)
    p1 = _agg128(h1, src, dst)
    h2 = _dense(p1, cntp, h1, Wl1, Wr1, b1, g2, be2)
    p2 = _agg128(h2, src, dst)
    h3, y = _dense_proj(p2, cntp, h2, Wl2, Wr2, b2, g3, be3, Wlo)
    p3 = _agg64(y, src, dst)
    return _final(p3, cntp, h3, batch.reshape(1, N), Wro, bo)


# speedup vs baseline: 3.1214x; 3.1214x over previous
import functools

import jax
import jax.numpy as jnp
from jax import lax
from jax.experimental import pallas as pl
from jax.experimental.pallas import tpu as pltpu
from jax.experimental.pallas import tpu_sc as plsc

N = 10000
D = 128
T = 64
G = 64

ACCROWS = 10112
C = 128
NCHUNK = 80
NW = 32
EP = NW * NCHUNK * C
TROWS = ACCROWS // 16

_mesh = plsc.VectorSubcoreMesh(core_axis_name="c", subcore_axis_name="s")


def _make_agg(d):
    @functools.partial(
        pl.kernel,
        out_type=jax.ShapeDtypeStruct((2 * ACCROWS, d), jnp.float32),
        mesh=_mesh,
        scratch_types=[
            pltpu.VMEM((NCHUNK // 2, C), jnp.int32),
            pltpu.VMEM((NCHUNK // 2, C), jnp.int32),
            pltpu.VMEM((C, d), jnp.float32),
            pltpu.VMEM((C, d), jnp.float32),
            pltpu.VMEM_SHARED((ACCROWS, d), jnp.float32),
            pltpu.SemaphoreType.DMA,
            pltpu.SemaphoreType.DMA,
        ],
    )
    def agg(table_hbm, src_hbm, dst_hbm, out_hbm, src_v, dst_v, bufa, bufb,
            acc, sga, sgb):
        cid = lax.axis_index("c")
        sid = lax.axis_index("s")
        wid = cid * 16 + sid
        half = NCHUNK // 2

        @pl.loop(0, C)
        def _(r):
            @pl.loop(0, d, step=16)
            def _(cc):
                bufa[r, pl.ds(cc, 16)] = jnp.zeros((16,), jnp.float32)

        r0 = sid * TROWS
        nfull = TROWS // C
        for k in range(nfull):
            pltpu.sync_copy(bufa, acc.at[pl.ds(r0 + k * C, C)])
        rem = TROWS - nfull * C
        if rem:
            pltpu.sync_copy(bufa.at[pl.ds(0, rem)],
                            acc.at[pl.ds(r0 + nfull * C, rem)])

        plsc.subcore_barrier()

        def start_gather(j, buf, sem):
            pltpu.make_async_copy(table_hbm.at[src_v.at[j]], buf, sem).start()

        def wait_gather(j, buf, sem):
            pltpu.make_async_copy(table_hbm.at[src_v.at[j]], buf, sem).wait()

        def scatter_add(j, buf):
            pltpu.sync_copy(buf, acc.at[dst_v.at[j]], add=True)

        for p in range(2):
            pltpu.sync_copy(src_hbm.at[wid, pl.ds(p * half, half)], src_v)
            pltpu.sync_copy(dst_hbm.at[wid, pl.ds(p * half, half)], dst_v)
            start_gather(0, bufa, sga)

            @pl.loop(0, half, step=2)
            def _(j):
                wait_gather(j, bufa, sga)
                start_gather(j + 1, bufb, sgb)
                scatter_add(j, bufa)
                wait_gather(j + 1, bufb, sgb)

                @pl.when(j + 2 < half)
                def _():
                    start_gather(j + 2, bufa, sga)

                scatter_add(j + 1, bufb)

        plsc.subcore_barrier()

        oo = pl.multiple_of(cid * ACCROWS + r0, 8)
        pltpu.sync_copy(acc.at[pl.ds(r0, TROWS)],
                        out_hbm.at[pl.ds(oo, TROWS)])

    return agg


@functools.partial(
    pl.kernel,
    out_type=jax.ShapeDtypeStruct((2 * ACCROWS, D), jnp.float32),
    mesh=_mesh,
    scratch_types=[
        pltpu.VMEM((NCHUNK, C), jnp.int32),
        pltpu.VMEM((C, D), jnp.float32),
        pltpu.VMEM_SHARED((ACCROWS, D), jnp.float32),
    ],
)
def _cnt_kernel(dst_hbm, out_hbm, dst_v, ones_v, acc):
    cid = lax.axis_index("c")
    sid = lax.axis_index("s")
    wid = cid * 16 + sid
    pltpu.sync_copy(dst_hbm.at[wid], dst_v)

    @pl.loop(0, C)
    def _(r):
        @pl.loop(0, D, step=16)
        def _(cc):
            ones_v[r, pl.ds(cc, 16)] = jnp.zeros((16,), jnp.float32)

    r0 = sid * TROWS
    nfull = TROWS // C
    for k in range(nfull):
        pltpu.sync_copy(ones_v, acc.at[pl.ds(r0 + k * C, C)])
    rem = TROWS - nfull * C
    if rem:
        pltpu.sync_copy(ones_v.at[pl.ds(0, rem)],
                        acc.at[pl.ds(r0 + nfull * C, rem)])

    @pl.loop(0, C)
    def _(r):
        @pl.loop(0, D, step=16)
        def _(cc):
            ones_v[r, pl.ds(cc, 16)] = jnp.ones((16,), jnp.float32)

    plsc.subcore_barrier()

    @pl.loop(0, NCHUNK)
    def _(j):
        pltpu.sync_copy(ones_v, acc.at[dst_v.at[j]], add=True)

    plsc.subcore_barrier()
    oo = pl.multiple_of(cid * ACCROWS + r0, 8)
    pltpu.sync_copy(acc.at[pl.ds(r0, TROWS)], out_hbm.at[pl.ds(oo, TROWS)])


_agg128 = _make_agg(D)


def _mean_from_partials(p_ref, cnt_ref):
    psum = p_ref[0:N, :] + p_ref[ACCROWS:ACCROWS + N, :]
    cnt = cnt_ref[0:N, 0:1] + cnt_ref[ACCROWS:ACCROWS + N, 0:1]
    return psum / jnp.maximum(cnt, 1.0)


def _dense_body(p_ref, cnt_ref, h_ref, wl_ref, wr_ref, b_ref, g_ref, be_ref,
                o_ref):
    mean = _mean_from_partials(p_ref, cnt_ref)
    z = (jnp.dot(mean, wl_ref[...], preferred_element_type=jnp.float32)
         + jnp.dot(h_ref[...], wr_ref[...], preferred_element_type=jnp.float32)
         + b_ref[...])
    mu = jnp.mean(z, axis=0, keepdims=True)
    var = jnp.mean((z - mu) * (z - mu), axis=0, keepdims=True)
    zn = (z - mu) / jnp.sqrt(var + 1e-5) * g_ref[...] + be_ref[...]
    o_ref[...] = jnp.maximum(zn, 0.0)


def _final_body(p_ref, cnt_ref, h_ref, batch_ref, wlo_ref, wro_ref, bo_ref,
                o_ref):
    s = _mean_from_partials(p_ref, cnt_ref)
    gi = lax.broadcasted_iota(jnp.int32, (G, N), 0)
    bm = (batch_ref[0:1, :] == gi).astype(jnp.float32)
    cg = jnp.sum(bm, axis=1, keepdims=True)
    pw = bm * jnp.where(cg > 0, 1.0 / jnp.maximum(cg, 1.0), 0.0)
    pooled_s = jnp.dot(pw, s, preferred_element_type=jnp.float32)
    pooled_h = jnp.dot(pw, h_ref[...], preferred_element_type=jnp.float32)
    o = (jnp.dot(pooled_s, wlo_ref[...], preferred_element_type=jnp.float32)
         + jnp.dot(pooled_h, wro_ref[...], preferred_element_type=jnp.float32)
         + bo_ref[...])
    o_ref[...] = jnp.where(cg > 0, o, 0.0)


def _dense(p, cntp, h, wl, wr, b, g, be):
    return pl.pallas_call(
        _dense_body,
        out_shape=jax.ShapeDtypeStruct((N, D), jnp.float32),
    )(p, cntp, h, wl, wr, b.reshape(1, -1), g.reshape(1, -1),
      be.reshape(1, -1))


def _final(p, cntp, h, batch2d, wlo, wro, bo):
    return pl.pallas_call(
        _final_body,
        out_shape=jax.ShapeDtypeStruct((G, T), jnp.float32),
    )(p, cntp, h, batch2d, wlo, wro, bo.reshape(1, -1))


def kernel(x, edge_index, batch, Wl0, Wr0, b0, g1, be1, Wl1, Wr1, b1, g2, be2,
           Wl2, Wr2, b2, g3, be3, Wlo, Wro, bo):
    e = edge_index.shape[1]
    pad = EP - e
    src = jnp.concatenate(
        [edge_index[0], jnp.zeros((pad,), jnp.int32)]).reshape(NW, NCHUNK, C)
    dst = jnp.concatenate(
        [edge_index[1], jnp.full((pad,), N, jnp.int32)]).reshape(NW, NCHUNK, C)

    cntp = _cnt_kernel(dst)
    p0 = _agg128(x, src, dst)
    h1 = _dense(p0, cntp, x, Wl0, Wr0, b0, g1, be1)
    p1 = _agg128(h1, src, dst)
    h2 = _dense(p1, cntp, h1, Wl1, Wr1, b1, g2, be2)
    p2 = _agg128(h2, src, dst)
    h3 = _dense(p2, cntp, h2, Wl2, Wr2, b2, g3, be3)
    p3 = _agg128(h3, src, dst)
    return _final(p3, cntp, h3, batch.reshape(1, N), Wlo, Wro, bo)


# speedup vs baseline: 9.4981x; 3.0429x over previous
import functools

import jax
import jax.numpy as jnp
from jax import lax
from jax.experimental import pallas as pl
from jax.experimental.pallas import tpu as pltpu
from jax.experimental.pallas import tpu_sc as plsc

N = 10000
D = 128
T = 64
G = 64

ACCROWS = 10112
C = 128
NCHUNK = 80
NW = 32
EP = NW * NCHUNK * C
TROWS = ACCROWS // 16

_mesh = plsc.VectorSubcoreMesh(core_axis_name="c", subcore_axis_name="s")


def _make_agg(d):
    @functools.partial(
        pl.kernel,
        out_type=jax.ShapeDtypeStruct((2 * ACCROWS, d), jnp.float32),
        mesh=_mesh,
        scratch_types=[
            pltpu.VMEM((NCHUNK // 2, C), jnp.int32),
            pltpu.VMEM((NCHUNK // 2, C), jnp.int32),
            pltpu.VMEM((C, d), jnp.float32),
            pltpu.VMEM((C, d), jnp.float32),
            pltpu.VMEM_SHARED((ACCROWS, d), jnp.float32),
            pltpu.SemaphoreType.DMA,
            pltpu.SemaphoreType.DMA,
        ],
    )
    def agg(table_hbm, src_hbm, dst_hbm, out_hbm, src_v, dst_v, bufa, bufb,
            acc, sga, sgb):
        cid = lax.axis_index("c")
        sid = lax.axis_index("s")
        wid = cid * 16 + sid
        half = NCHUNK // 2

        @pl.loop(0, C)
        def _(r):
            @pl.loop(0, d, step=16)
            def _(cc):
                bufa[r, pl.ds(cc, 16)] = jnp.zeros((16,), jnp.float32)

        r0 = sid * TROWS
        nfull = TROWS // C
        for k in range(nfull):
            pltpu.sync_copy(bufa, acc.at[pl.ds(r0 + k * C, C)])
        rem = TROWS - nfull * C
        if rem:
            pltpu.sync_copy(bufa.at[pl.ds(0, rem)],
                            acc.at[pl.ds(r0 + nfull * C, rem)])

        plsc.subcore_barrier()

        def start_gather(j, buf, sem):
            pltpu.make_async_copy(table_hbm.at[src_v.at[j]], buf, sem).start()

        def wait_gather(j, buf, sem):
            pltpu.make_async_copy(table_hbm.at[src_v.at[j]], buf, sem).wait()

        def scatter_add(j, buf):
            pltpu.sync_copy(buf, acc.at[dst_v.at[j]], add=True)

        for p in range(2):
            pltpu.sync_copy(src_hbm.at[wid, pl.ds(p * half, half)], src_v)
            pltpu.sync_copy(dst_hbm.at[wid, pl.ds(p * half, half)], dst_v)
            start_gather(0, bufa, sga)

            @pl.loop(0, half, step=2)
            def _(j):
                wait_gather(j, bufa, sga)
                start_gather(j + 1, bufb, sgb)
                scatter_add(j, bufa)
                wait_gather(j + 1, bufb, sgb)

                @pl.when(j + 2 < half)
                def _():
                    start_gather(j + 2, bufa, sga)

                scatter_add(j + 1, bufb)

        plsc.subcore_barrier()

        oo = pl.multiple_of(cid * ACCROWS + r0, 8)
        pltpu.sync_copy(acc.at[pl.ds(r0, TROWS)],
                        out_hbm.at[pl.ds(oo, TROWS)])

    return agg


@functools.partial(
    pl.kernel,
    out_type=jax.ShapeDtypeStruct((2 * ACCROWS, D), jnp.float32),
    mesh=_mesh,
    scratch_types=[
        pltpu.VMEM((NCHUNK, C), jnp.int32),
        pltpu.VMEM((C, D), jnp.float32),
        pltpu.VMEM_SHARED((ACCROWS, D), jnp.float32),
    ],
)
def _cnt_kernel(dst_hbm, out_hbm, dst_v, ones_v, acc):
    cid = lax.axis_index("c")
    sid = lax.axis_index("s")
    wid = cid * 16 + sid
    pltpu.sync_copy(dst_hbm.at[wid], dst_v)

    @pl.loop(0, C)
    def _(r):
        @pl.loop(0, D, step=16)
        def _(cc):
            ones_v[r, pl.ds(cc, 16)] = jnp.zeros((16,), jnp.float32)

    r0 = sid * TROWS
    nfull = TROWS // C
    for k in range(nfull):
        pltpu.sync_copy(ones_v, acc.at[pl.ds(r0 + k * C, C)])
    rem = TROWS - nfull * C
    if rem:
        pltpu.sync_copy(ones_v.at[pl.ds(0, rem)],
                        acc.at[pl.ds(r0 + nfull * C, rem)])

    @pl.loop(0, C)
    def _(r):
        @pl.loop(0, D, step=16)
        def _(cc):
            ones_v[r, pl.ds(cc, 16)] = jnp.ones((16,), jnp.float32)

    plsc.subcore_barrier()

    @pl.loop(0, NCHUNK)
    def _(j):
        pltpu.sync_copy(ones_v, acc.at[dst_v.at[j]], add=True)

    plsc.subcore_barrier()
    oo = pl.multiple_of(cid * ACCROWS + r0, 8)
    pltpu.sync_copy(acc.at[pl.ds(r0, TROWS)], out_hbm.at[pl.ds(oo, TROWS)])


_agg128 = _make_agg(D)


def _mean_from_partials(p_ref, cnt_ref):
    psum = p_ref[0:N, :] + p_ref[ACCROWS:ACCROWS + N, :]
    cnt = cnt_ref[0:N, 0:1] + cnt_ref[ACCROWS:ACCROWS + N, 0:1]
    return psum / jnp.maximum(cnt, 1.0)


def _dense_body(p_ref, cnt_ref, h_ref, wl_ref, wr_ref, b_ref, g_ref, be_ref,
                o_ref):
    mean = _mean_from_partials(p_ref, cnt_ref)
    z = (jnp.dot(mean, wl_ref[...], preferred_element_type=jnp.float32)
         + jnp.dot(h_ref[...], wr_ref[...], preferred_element_type=jnp.float32)
         + b_ref[...])
    mu = jnp.mean(z, axis=0, keepdims=True)
    var = jnp.mean((z - mu) * (z - mu), axis=0, keepdims=True)
    zn = (z - mu) / jnp.sqrt(var + 1e-5) * g_ref[...] + be_ref[...]
    o_ref[...] = jnp.maximum(zn, 0.0)


def _final_body(p_ref, cnt_ref, h_ref, batch_ref, wlo_ref, wro_ref, bo_ref,
                o_ref):
    s = _mean_from_partials(p_ref, cnt_ref)
    gi = lax.broadcasted_iota(jnp.int32, (G, N), 0)
    bm = (batch_ref[0:1, :] == gi).astype(jnp.float32)
    cg = jnp.sum(bm, axis=1, keepdims=True)
    pw = bm * jnp.where(cg > 0, 1.0 / jnp.maximum(cg, 1.0), 0.0)
    pooled_s = jnp.dot(pw, s, preferred_element_type=jnp.float32)
    pooled_h = jnp.dot(pw, h_ref[...], preferred_element_type=jnp.float32)
    o = (jnp.dot(pooled_s, wlo_ref[...], preferred_element_type=jnp.float32)
         + jnp.dot(pooled_h, wro_ref[...], preferred_element_type=jnp.float32)
         + bo_ref[...])
    o_ref[...] = jnp.where(cg > 0, o, 0.0)


def _dense(p, cntp, h, wl, wr, b, g, be):
    return pl.pallas_call(
        _dense_body,
        out_shape=jax.ShapeDtypeStruct((N, D), jnp.float32),
    )(p, cntp, h, wl, wr, b.reshape(1, -1), g.reshape(1, -1),
      be.reshape(1, -1))


def _final(p, cntp, h, batch2d, wlo, wro, bo):
    return pl.pallas_call(
        _final_body,
        out_shape=jax.ShapeDtypeStruct((G, T), jnp.float32),
    )(p, cntp, h, batch2d, wlo, wro, bo.reshape(1, -1))


def kernel(x, edge_index, batch, Wl0, Wr0, b0, g1, be1, Wl1, Wr1, b1, g2, be2,
           Wl2, Wr2, b2, g3, be3, Wlo, Wro, bo):
    e = edge_index.shape[1]
    pad = EP - e
    pidx = jnp.arange(pad, dtype=jnp.int32)
    src = jnp.concatenate(
        [edge_index[0], pidx % N]).reshape(NW, NCHUNK, C)
    dst = jnp.concatenate(
        [edge_index[1], N + pidx % (ACCROWS - N)]).reshape(NW, NCHUNK, C)

    cntp = _cnt_kernel(dst)
    p0 = _agg128(x, src, dst)
    h1 = _dense(p0, cntp, x, Wl0, Wr0, b0, g1, be1)
    p1 = _agg128(h1, src, dst)
    h2 = _dense(p1, cntp, h1, Wl1, Wr1, b1, g2, be2)
    p2 = _agg128(h2, src, dst)
    h3 = _dense(p2, cntp, h2, Wl2, Wr2, b2, g3, be3)
    p3 = _agg128(h3, src, dst)
    return _final(p3, cntp, h3, batch.reshape(1, N), Wlo, Wro, bo)
